# Initial kernel scaffold; baseline (speedup 1.0000x reference)
import jax
import jax.numpy as jnp
from jax.experimental import pallas as pl


def kernel(x, W1, a1_s, a1_d, b1, W2, a2_s, a2_d, b2, W3, a3_s, a3_d, b3, Wl, bl, edge_index, batch):
    raise NotImplementedError("write your pallas kernel here")


# speedup vs baseline: 6.2453x; 6.2453x over previous
import functools

import jax
import jax.numpy as jnp
from jax import lax
from jax.experimental import pallas as pl
from jax.experimental.pallas import tpu as pltpu
from jax.experimental.pallas import tpu_sc as plsc

N = 10000
E = 160000
HID = 64
G = 128

NW = 32
CH = 2000
NCHUNK = E // CH
VREG = CH // 16


def _prep_first_body(x_ref, w_ref, as_ref, ad_ref, ht_ref, ls_ref, ld_ref):
    x = x_ref[...]
    w = w_ref[...]
    ht = lax.dot_general(w, x, (((0,), (1,)), ((), ())),
                         preferred_element_type=jnp.float32)
    ht_ref[...] = ht
    ls_ref[...] = lax.dot_general(as_ref[...], ht, (((1,), (0,)), ((), ())),
                                  preferred_element_type=jnp.float32)
    ld_ref[...] = lax.dot_general(ad_ref[...], ht, (((1,), (0,)), ((), ())),
                                  preferred_element_type=jnp.float32)


def _prep_mid_body(prev_ref, b_ref, w_ref, as_ref, ad_ref,
                   ht_ref, ls_ref, ld_ref):
    xt = jnp.maximum(prev_ref[...] + b_ref[...], 0.0)
    w = w_ref[...]
    ht = lax.dot_general(w, xt, (((0,), (0,)), ((), ())),
                         preferred_element_type=jnp.float32)
    ht_ref[...] = ht
    ls_ref[...] = lax.dot_general(as_ref[...], ht, (((1,), (0,)), ((), ())),
                                  preferred_element_type=jnp.float32)
    ld_ref[...] = lax.dot_general(ad_ref[...], ht, (((1,), (0,)), ((), ())),
                                  preferred_element_type=jnp.float32)


_prep_out = [
    jax.ShapeDtypeStruct((HID, N), jnp.float32),
    jax.ShapeDtypeStruct((1, N), jnp.float32),
    jax.ShapeDtypeStruct((1, N), jnp.float32),
]


def _prep_first(x, w, a_s, a_d):
    return pl.pallas_call(_prep_first_body, out_shape=_prep_out)(
        x, w, a_s.reshape(1, HID), a_d.reshape(1, HID))


def _prep_mid(prev_t, b, w, a_s, a_d):
    return pl.pallas_call(_prep_mid_body, out_shape=_prep_out)(
        prev_t, b.reshape(HID, 1), w, a_s.reshape(1, HID), a_d.reshape(1, HID))


def _sc_edge_body(ht_hbm, ls_hbm, ld_hbm, src_hbm, dst_hbm, out_hbm,
                  ls_v, ld_v, s_v, h0_v, h1_v, o0_v, o1_v, src_v, dst_v):
    wid = lax.axis_index("s") * 2 + lax.axis_index("c")
    r0 = 2 * wid

    pltpu.sync_copy(ls_hbm, ls_v)
    pltpu.sync_copy(ld_hbm, ld_v)
    pltpu.sync_copy(ht_hbm.at[r0], h0_v)
    pltpu.sync_copy(ht_hbm.at[r0 + 1], h1_v)

    def _zero(i, _):
        sl = pl.ds(i * 16, 16)
        z = jnp.zeros((16,), jnp.float32)
        s_v[sl] = z
        o0_v[sl] = z
        o1_v[sl] = z
        return 0

    lax.fori_loop(0, N // 16, _zero, 0)

    def _edge_w(j):
        sl = pl.ds(j * 16, 16)
        sidx = src_v[sl]
        didx = dst_v[sl]
        z = plsc.load_gather(ls_v, [sidx]) + plsc.load_gather(ld_v, [didx])
        z = jnp.maximum(z, 0.2 * z)
        return sidx, didx, jnp.exp(z)

    def _pass1_chunk(c, _):
        off = pl.ds(c * CH, CH)
        pltpu.sync_copy(src_hbm.at[off], src_v)
        pltpu.sync_copy(dst_hbm.at[off], dst_v)

        def _body(j, _):
            _, didx, w = _edge_w(j)
            plsc.addupdate_scatter(s_v, [didx], w)
            return 0

        lax.fori_loop(0, VREG, _body, 0)
        return 0

    lax.fori_loop(0, NCHUNK, _pass1_chunk, 0)

    def _pass2_chunk(c, _):
        off = pl.ds(c * CH, CH)
        pltpu.sync_copy(src_hbm.at[off], src_v)
        pltpu.sync_copy(dst_hbm.at[off], dst_v)

        def _body(j, _):
            sidx, didx, w = _edge_w(j)
            coef = w / (plsc.load_gather(s_v, [didx]) + 1e-16)
            g0 = plsc.load_gather(h0_v, [sidx])
            plsc.addupdate_scatter(o0_v, [didx], g0 * coef)
            g1 = plsc.load_gather(h1_v, [sidx])
            plsc.addupdate_scatter(o1_v, [didx], g1 * coef)
            return 0

        lax.fori_loop(0, VREG, _body, 0)
        return 0

    lax.fori_loop(0, NCHUNK, _pass2_chunk, 0)

    pltpu.sync_copy(o0_v, out_hbm.at[r0])
    pltpu.sync_copy(o1_v, out_hbm.at[r0 + 1])


def _sc_edge(ht, ls, ld, src, dst):
    mesh = plsc.VectorSubcoreMesh(core_axis_name="c", subcore_axis_name="s")
    k = functools.partial(
        pl.kernel,
        mesh=mesh,
        compiler_params=pltpu.CompilerParams(needs_layout_passes=False),
        out_type=jax.ShapeDtypeStruct((HID, N), jnp.float32),
        scratch_types=[
            pltpu.VMEM((N,), jnp.float32),
            pltpu.VMEM((N,), jnp.float32),
            pltpu.VMEM((N,), jnp.float32),
            pltpu.VMEM((N,), jnp.float32),
            pltpu.VMEM((N,), jnp.float32),
            pltpu.VMEM((N,), jnp.float32),
            pltpu.VMEM((N,), jnp.float32),
            pltpu.VMEM((CH,), jnp.int32),
            pltpu.VMEM((CH,), jnp.int32),
        ],
    )(_sc_edge_body)
    return k(ht, ls.reshape(N), ld.reshape(N), src, dst)


def _pool_body(ht_ref, b_ref, batch_ref, wl_ref, bl_ref, out_ref, mx_ref):
    h3 = ht_ref[...] + b_ref[...]
    bvec = batch_ref[...]
    gids = lax.broadcasted_iota(jnp.int32, (G, N), 0)
    oh = (gids == bvec).astype(jnp.float32)
    sums = lax.dot_general(oh, h3, (((1,), (1,)), ((), ())),
                           preferred_element_type=jnp.float32)
    cnt = jnp.sum(oh, axis=1, keepdims=True)
    mean = sums / jnp.maximum(cnt, 1.0)

    neg = jnp.float32(-jnp.inf)

    def _gmax(g, _):
        mask = bvec == g
        hm = jnp.where(mask, h3, neg)
        mg = jnp.max(hm, axis=1)
        mx_ref[pl.ds(g, 1), :] = mg.reshape(1, HID)
        return 0

    lax.fori_loop(0, G, _gmax, 0)
    mx = mx_ref[...]
    mx = jnp.where(mx == neg, 0.0, mx)
    pooled = jnp.concatenate([mean, mx], axis=1)
    out_ref[...] = lax.dot_general(pooled, wl_ref[...],
                                   (((1,), (0,)), ((), ())),
                                   preferred_element_type=jnp.float32) \
        + bl_ref[...]


def _pool(ht, b, batch, wl, bl):
    out, _ = pl.pallas_call(
        _pool_body,
        out_shape=[jax.ShapeDtypeStruct((G, 2), jnp.float32),
                   jax.ShapeDtypeStruct((G, HID), jnp.float32)],
    )(ht, b.reshape(HID, 1), batch.reshape(1, N), wl, bl.reshape(1, 2))
    return out


@jax.jit
def kernel(x, W1, a1_s, a1_d, b1, W2, a2_s, a2_d, b2, W3, a3_s, a3_d, b3,
           Wl, bl, edge_index, batch):
    src = edge_index[0]
    dst = edge_index[1]
    ht, ls, ld = _prep_first(x, W1, a1_s, a1_d)
    o1 = _sc_edge(ht, ls, ld, src, dst)
    ht, ls, ld = _prep_mid(o1, b1, W2, a2_s, a2_d)
    o2 = _sc_edge(ht, ls, ld, src, dst)
    ht, ls, ld = _prep_mid(o2, b2, W3, a3_s, a3_d)
    o3 = _sc_edge(ht, ls, ld, src, dst)
    return _pool(o3, b3, batch, Wl, bl)


# speedup vs baseline: 10.0082x; 1.6025x over previous
import functools

import jax
import jax.numpy as jnp
from jax import lax
from jax.experimental import pallas as pl
from jax.experimental.pallas import tpu as pltpu
from jax.experimental.pallas import tpu_sc as plsc

N = 10000
E = 160000
HID = 64
G = 128

NW = 32
CH = 2000
NCHUNK = E // CH
VREG = CH // 16


def _prep_first_body(x_ref, w_ref, as_ref, ad_ref, ht_ref, ls_ref, ld_ref):
    x = x_ref[...]
    w = w_ref[...]
    ht = lax.dot_general(w, x, (((0,), (1,)), ((), ())),
                         preferred_element_type=jnp.float32)
    ht_ref[...] = ht
    ls_ref[...] = lax.dot_general(as_ref[...], ht, (((1,), (0,)), ((), ())),
                                  preferred_element_type=jnp.float32)
    ld_ref[...] = lax.dot_general(ad_ref[...], ht, (((1,), (0,)), ((), ())),
                                  preferred_element_type=jnp.float32)


def _prep_mid_body(prev_ref, b_ref, w_ref, as_ref, ad_ref,
                   ht_ref, ls_ref, ld_ref):
    xt = jnp.maximum(prev_ref[...] + b_ref[...], 0.0)
    w = w_ref[...]
    ht = lax.dot_general(w, xt, (((0,), (0,)), ((), ())),
                         preferred_element_type=jnp.float32)
    ht_ref[...] = ht
    ls_ref[...] = lax.dot_general(as_ref[...], ht, (((1,), (0,)), ((), ())),
                                  preferred_element_type=jnp.float32)
    ld_ref[...] = lax.dot_general(ad_ref[...], ht, (((1,), (0,)), ((), ())),
                                  preferred_element_type=jnp.float32)


_prep_out = [
    jax.ShapeDtypeStruct((HID, N), jnp.float32),
    jax.ShapeDtypeStruct((1, N), jnp.float32),
    jax.ShapeDtypeStruct((1, N), jnp.float32),
]


def _prep_first(x, w, a_s, a_d):
    return pl.pallas_call(_prep_first_body, out_shape=_prep_out)(
        x, w, a_s.reshape(1, HID), a_d.reshape(1, HID))


def _prep_mid(prev_t, b, w, a_s, a_d):
    return pl.pallas_call(_prep_mid_body, out_shape=_prep_out)(
        prev_t, b.reshape(HID, 1), w, a_s.reshape(1, HID), a_d.reshape(1, HID))


def _sc_edge_body(ht_hbm, ls_hbm, ld_hbm, src_hbm, dst_hbm, out_hbm,
                  ls_v, ld_v, s_v, h0_v, h1_v, o0_v, o1_v, src_v, dst_v,
                  idx_v, s_sh):
    sid = lax.axis_index("s")
    wid = sid * 2 + lax.axis_index("c")
    r0 = 2 * wid

    pltpu.sync_copy(ls_hbm, ls_v)
    pltpu.sync_copy(ld_hbm, ld_v)
    pltpu.sync_copy(ht_hbm.at[r0], h0_v)
    pltpu.sync_copy(ht_hbm.at[r0 + 1], h1_v)

    def _zero(i, _):
        sl = pl.ds(i * 16, 16)
        z = jnp.zeros((16,), jnp.float32)
        s_v[sl] = z
        o0_v[sl] = z
        o1_v[sl] = z
        idx_v[sl] = lax.iota(jnp.int32, 16) + i * 16
        return 0

    lax.fori_loop(0, N // 16, _zero, 0)

    @pl.when(sid == 0)
    def _():
        pltpu.sync_copy(o0_v, s_sh)

    plsc.subcore_barrier()

    def _edge_w(j):
        sl = pl.ds(j * 16, 16)
        sidx = src_v[sl]
        didx = dst_v[sl]
        z = plsc.load_gather(ls_v, [sidx]) + plsc.load_gather(ld_v, [didx])
        z = jnp.maximum(z, 0.2 * z)
        return sidx, didx, jnp.exp(z)

    ebase = sid * (E // 16)

    def _pass1_chunk(c, _):
        off = pl.ds(ebase + c * CH, CH)
        pltpu.sync_copy(src_hbm.at[off], src_v)
        pltpu.sync_copy(dst_hbm.at[off], dst_v)

        def _body(j, _):
            _, didx, w = _edge_w(j)
            plsc.addupdate_scatter(s_v, [didx], w)
            return 0

        lax.fori_loop(0, VREG, _body, 0)
        return 0

    lax.fori_loop(0, E // 16 // CH, _pass1_chunk, 0)
    pltpu.sync_copy(s_v, s_sh.at[idx_v], add=True)
    plsc.subcore_barrier()
    pltpu.sync_copy(s_sh, s_v)

    def _pass2_chunk(c, _):
        off = pl.ds(c * CH, CH)
        pltpu.sync_copy(src_hbm.at[off], src_v)
        pltpu.sync_copy(dst_hbm.at[off], dst_v)

        def _body(j, _):
            sidx, didx, w = _edge_w(j)
            coef = w / (plsc.load_gather(s_v, [didx]) + 1e-16)
            g0 = plsc.load_gather(h0_v, [sidx])
            plsc.addupdate_scatter(o0_v, [didx], g0 * coef)
            g1 = plsc.load_gather(h1_v, [sidx])
            plsc.addupdate_scatter(o1_v, [didx], g1 * coef)
            return 0

        lax.fori_loop(0, VREG, _body, 0)
        return 0

    lax.fori_loop(0, NCHUNK, _pass2_chunk, 0)

    pltpu.sync_copy(o0_v, out_hbm.at[r0])
    pltpu.sync_copy(o1_v, out_hbm.at[r0 + 1])


def _sc_edge(ht, ls, ld, src, dst):
    mesh = plsc.VectorSubcoreMesh(core_axis_name="c", subcore_axis_name="s")
    k = functools.partial(
        pl.kernel,
        mesh=mesh,
        compiler_params=pltpu.CompilerParams(needs_layout_passes=False),
        out_type=jax.ShapeDtypeStruct((HID, N), jnp.float32),
        scratch_types=[
            pltpu.VMEM((N,), jnp.float32),
            pltpu.VMEM((N,), jnp.float32),
            pltpu.VMEM((N,), jnp.float32),
            pltpu.VMEM((N,), jnp.float32),
            pltpu.VMEM((N,), jnp.float32),
            pltpu.VMEM((N,), jnp.float32),
            pltpu.VMEM((N,), jnp.float32),
            pltpu.VMEM((CH,), jnp.int32),
            pltpu.VMEM((CH,), jnp.int32),
            pltpu.VMEM((N,), jnp.int32),
            pltpu.VMEM_SHARED((N,), jnp.float32),
        ],
    )(_sc_edge_body)
    return k(ht, ls.reshape(N), ld.reshape(N), src, dst)


def _pool_body(ht_ref, b_ref, batch_ref, wl_ref, bl_ref, out_ref, mx_ref):
    h3 = ht_ref[...] + b_ref[...]
    bvec = batch_ref[...]
    gids = lax.broadcasted_iota(jnp.int32, (G, N), 0)
    oh = (gids == bvec).astype(jnp.float32)
    sums = lax.dot_general(oh, h3, (((1,), (1,)), ((), ())),
                           preferred_element_type=jnp.float32)
    cnt = jnp.sum(oh, axis=1, keepdims=True)
    mean = sums / jnp.maximum(cnt, 1.0)

    neg = jnp.float32(-jnp.inf)

    def _gmax(g, _):
        mask = bvec == g
        hm = jnp.where(mask, h3, neg)
        mg = jnp.max(hm, axis=1)
        mx_ref[pl.ds(g, 1), :] = mg.reshape(1, HID)
        return 0

    lax.fori_loop(0, G, _gmax, 0)
    mx = mx_ref[...]
    mx = jnp.where(mx == neg, 0.0, mx)
    pooled = jnp.concatenate([mean, mx], axis=1)
    out_ref[...] = lax.dot_general(pooled, wl_ref[...],
                                   (((1,), (0,)), ((), ())),
                                   preferred_element_type=jnp.float32) \
        + bl_ref[...]


def _pool(ht, b, batch, wl, bl):
    out, _ = pl.pallas_call(
        _pool_body,
        out_shape=[jax.ShapeDtypeStruct((G, 2), jnp.float32),
                   jax.ShapeDtypeStruct((G, HID), jnp.float32)],
    )(ht, b.reshape(HID, 1), batch.reshape(1, N), wl, bl.reshape(1, 2))
    return out


@jax.jit
def kernel(x, W1, a1_s, a1_d, b1, W2, a2_s, a2_d, b2, W3, a3_s, a3_d, b3,
           Wl, bl, edge_index, batch):
    src = edge_index[0]
    dst = edge_index[1]
    ht, ls, ld = _prep_first(x, W1, a1_s, a1_d)
    o1 = _sc_edge(ht, ls, ld, src, dst)
    ht, ls, ld = _prep_mid(o1, b1, W2, a2_s, a2_d)
    o2 = _sc_edge(ht, ls, ld, src, dst)
    ht, ls, ld = _prep_mid(o2, b2, W3, a3_s, a3_d)
    o3 = _sc_edge(ht, ls, ld, src, dst)
    return _pool(o3, b3, batch, Wl, bl)


# speedup vs baseline: 12.1652x; 1.2155x over previous
import functools

import jax
import jax.numpy as jnp
from jax import lax
from jax.experimental import pallas as pl
from jax.experimental.pallas import tpu as pltpu
from jax.experimental.pallas import tpu_sc as plsc

N = 10000
E = 160000
HID = 64
G = 128

NW = 32
CH = 2000
NCHUNK = E // CH
VREG = CH // 16


def _prep_first_body(x_ref, w_ref, as_ref, ad_ref, ht_ref, ls_ref, ld_ref):
    x = x_ref[...]
    w = w_ref[...]
    ht = lax.dot_general(w, x, (((0,), (1,)), ((), ())),
                         preferred_element_type=jnp.float32)
    ht_ref[...] = ht
    ls_ref[...] = lax.dot_general(as_ref[...], ht, (((1,), (0,)), ((), ())),
                                  preferred_element_type=jnp.float32)
    ld_ref[...] = lax.dot_general(ad_ref[...], ht, (((1,), (0,)), ((), ())),
                                  preferred_element_type=jnp.float32)


def _prep_mid_body(prev_ref, b_ref, w_ref, as_ref, ad_ref,
                   ht_ref, ls_ref, ld_ref):
    xt = jnp.maximum(prev_ref[...] + b_ref[...], 0.0)
    w = w_ref[...]
    ht = lax.dot_general(w, xt, (((0,), (0,)), ((), ())),
                         preferred_element_type=jnp.float32)
    ht_ref[...] = ht
    ls_ref[...] = lax.dot_general(as_ref[...], ht, (((1,), (0,)), ((), ())),
                                  preferred_element_type=jnp.float32)
    ld_ref[...] = lax.dot_general(ad_ref[...], ht, (((1,), (0,)), ((), ())),
                                  preferred_element_type=jnp.float32)


_prep_out = [
    jax.ShapeDtypeStruct((HID, N), jnp.float32),
    jax.ShapeDtypeStruct((1, N), jnp.float32),
    jax.ShapeDtypeStruct((1, N), jnp.float32),
]


def _prep_first(x, w, a_s, a_d):
    return pl.pallas_call(_prep_first_body, out_shape=_prep_out)(
        x, w, a_s.reshape(1, HID), a_d.reshape(1, HID))


def _prep_mid(prev_t, b, w, a_s, a_d):
    return pl.pallas_call(_prep_mid_body, out_shape=_prep_out)(
        prev_t, b.reshape(HID, 1), w, a_s.reshape(1, HID), a_d.reshape(1, HID))


def _sc_edge_body(ht_hbm, ls_hbm, ld_hbm, src_hbm, dst_hbm, out_hbm,
                  ls_v, ld_v, s_v, h0_v, h1_v, o0_v, o1_v, src_v, dst_v,
                  idx_v, coef_v, s_sh, coef_sh):
    sid = lax.axis_index("s")
    wid = sid * 2 + lax.axis_index("c")
    r0 = 2 * wid

    pltpu.sync_copy(ls_hbm, ls_v)
    pltpu.sync_copy(ld_hbm, ld_v)
    pltpu.sync_copy(ht_hbm.at[r0], h0_v)
    pltpu.sync_copy(ht_hbm.at[r0 + 1], h1_v)

    def _zero(i, _):
        sl = pl.ds(i * 16, 16)
        z = jnp.zeros((16,), jnp.float32)
        s_v[sl] = z
        o0_v[sl] = z
        o1_v[sl] = z
        idx_v[sl] = lax.iota(jnp.int32, 16) + i * 16
        return 0

    lax.fori_loop(0, N // 16, _zero, 0)

    @pl.when(sid == 0)
    def _():
        pltpu.sync_copy(o0_v, s_sh)

    plsc.subcore_barrier()

    def _edge_w(j):
        sl = pl.ds(j * 16, 16)
        sidx = src_v[sl]
        didx = dst_v[sl]
        z = plsc.load_gather(ls_v, [sidx]) + plsc.load_gather(ld_v, [didx])
        z = jnp.maximum(z, 0.2 * z)
        return sidx, didx, jnp.exp(z)

    ebase = sid * (E // 16)

    def _pass1_chunk(c, _):
        off = pl.ds(ebase + c * CH, CH)
        pltpu.sync_copy(src_hbm.at[off], src_v)
        pltpu.sync_copy(dst_hbm.at[off], dst_v)

        def _body(j, _):
            _, didx, w = _edge_w(j)
            plsc.addupdate_scatter(s_v, [didx], w)
            return 0

        lax.fori_loop(0, VREG, _body, 0)
        return 0

    lax.fori_loop(0, E // 16 // CH, _pass1_chunk, 0)
    pltpu.sync_copy(s_v, s_sh.at[idx_v], add=True)
    plsc.subcore_barrier()
    pltpu.sync_copy(s_sh, s_v)

    def _coef_chunk(c, _):
        off = pl.ds(ebase + c * CH, CH)
        pltpu.sync_copy(src_hbm.at[off], src_v)
        pltpu.sync_copy(dst_hbm.at[off], dst_v)

        def _body(j, _):
            sl = pl.ds(j * 16, 16)
            _, didx, w = _edge_w(j)
            coef_v[sl] = w / (plsc.load_gather(s_v, [didx]) + 1e-16)
            return 0

        lax.fori_loop(0, VREG, _body, 0)
        pltpu.sync_copy(coef_v, coef_sh.at[off])
        return 0

    lax.fori_loop(0, E // 16 // CH, _coef_chunk, 0)
    plsc.subcore_barrier()

    def _pass2_chunk(c, _):
        off = pl.ds(c * CH, CH)
        pltpu.sync_copy(src_hbm.at[off], src_v)
        pltpu.sync_copy(dst_hbm.at[off], dst_v)
        pltpu.sync_copy(coef_sh.at[off], coef_v)

        def _body(j, _):
            sl = pl.ds(j * 16, 16)
            sidx = src_v[sl]
            didx = dst_v[sl]
            coef = coef_v[sl]
            g0 = plsc.load_gather(h0_v, [sidx])
            plsc.addupdate_scatter(o0_v, [didx], g0 * coef)
            g1 = plsc.load_gather(h1_v, [sidx])
            plsc.addupdate_scatter(o1_v, [didx], g1 * coef)
            return 0

        lax.fori_loop(0, VREG, _body, 0)
        return 0

    lax.fori_loop(0, NCHUNK, _pass2_chunk, 0)

    pltpu.sync_copy(o0_v, out_hbm.at[r0])
    pltpu.sync_copy(o1_v, out_hbm.at[r0 + 1])


def _sc_edge(ht, ls, ld, src, dst):
    mesh = plsc.VectorSubcoreMesh(core_axis_name="c", subcore_axis_name="s")
    k = functools.partial(
        pl.kernel,
        mesh=mesh,
        compiler_params=pltpu.CompilerParams(needs_layout_passes=False),
        out_type=jax.ShapeDtypeStruct((HID, N), jnp.float32),
        scratch_types=[
            pltpu.VMEM((N,), jnp.float32),
            pltpu.VMEM((N,), jnp.float32),
            pltpu.VMEM((N,), jnp.float32),
            pltpu.VMEM((N,), jnp.float32),
            pltpu.VMEM((N,), jnp.float32),
            pltpu.VMEM((N,), jnp.float32),
            pltpu.VMEM((N,), jnp.float32),
            pltpu.VMEM((CH,), jnp.int32),
            pltpu.VMEM((CH,), jnp.int32),
            pltpu.VMEM((N,), jnp.int32),
            pltpu.VMEM((CH,), jnp.float32),
            pltpu.VMEM_SHARED((N,), jnp.float32),
            pltpu.VMEM_SHARED((E,), jnp.float32),
        ],
    )(_sc_edge_body)
    return k(ht, ls.reshape(N), ld.reshape(N), src, dst)


def _pool_body(ht_ref, b_ref, batch_ref, wl_ref, bl_ref, out_ref, mx_ref):
    h3 = ht_ref[...] + b_ref[...]
    bvec = batch_ref[...]
    gids = lax.broadcasted_iota(jnp.int32, (G, N), 0)
    oh = (gids == bvec).astype(jnp.float32)
    sums = lax.dot_general(oh, h3, (((1,), (1,)), ((), ())),
                           preferred_element_type=jnp.float32)
    cnt = jnp.sum(oh, axis=1, keepdims=True)
    mean = sums / jnp.maximum(cnt, 1.0)

    neg = jnp.float32(-jnp.inf)

    def _gmax(g, _):
        mask = bvec == g
        hm = jnp.where(mask, h3, neg)
        mg = jnp.max(hm, axis=1)
        mx_ref[pl.ds(g, 1), :] = mg.reshape(1, HID)
        return 0

    lax.fori_loop(0, G, _gmax, 0)
    mx = mx_ref[...]
    mx = jnp.where(mx == neg, 0.0, mx)
    pooled = jnp.concatenate([mean, mx], axis=1)
    out_ref[...] = lax.dot_general(pooled, wl_ref[...],
                                   (((1,), (0,)), ((), ())),
                                   preferred_element_type=jnp.float32) \
        + bl_ref[...]


def _pool(ht, b, batch, wl, bl):
    out, _ = pl.pallas_call(
        _pool_body,
        out_shape=[jax.ShapeDtypeStruct((G, 2), jnp.float32),
                   jax.ShapeDtypeStruct((G, HID), jnp.float32)],
    )(ht, b.reshape(HID, 1), batch.reshape(1, N), wl, bl.reshape(1, 2))
    return out


@jax.jit
def kernel(x, W1, a1_s, a1_d, b1, W2, a2_s, a2_d, b2, W3, a3_s, a3_d, b3,
           Wl, bl, edge_index, batch):
    src = edge_index[0]
    dst = edge_index[1]
    ht, ls, ld = _prep_first(x, W1, a1_s, a1_d)
    o1 = _sc_edge(ht, ls, ld, src, dst)
    ht, ls, ld = _prep_mid(o1, b1, W2, a2_s, a2_d)
    o2 = _sc_edge(ht, ls, ld, src, dst)
    ht, ls, ld = _prep_mid(o2, b2, W3, a3_s, a3_d)
    o3 = _sc_edge(ht, ls, ld, src, dst)
    return _pool(o3, b3, batch, Wl, bl)
